# baseline (speedup 1.0000x reference)
import functools

import jax
import jax.numpy as jnp
from jax import lax
from jax.experimental import pallas as pl
from jax.experimental.pallas import tpu as pltpu
from jax.experimental.pallas import tpu_sc as plsc

N = 100000
D = 128
B = 16384
TB = 3 * B
K = 20
H = 2
DH = D // H


_CRB = 1000


def _combine_body(nf_ref, mem_ref, out_ref):
    s = (nf_ref[...] + mem_ref[...]).astype(jnp.bfloat16).astype(jnp.float32)
    bits = lax.bitcast_convert_type(s, jnp.int32)
    lo = lax.shift_right_logical(bits[:, :D // 2], 16)
    hi = jnp.bitwise_and(bits[:, D // 2:], jnp.int32(-65536))
    out_ref[...] = jnp.bitwise_or(lo, hi)


def _combine(node_feat, memory):
    return pl.pallas_call(
        _combine_body,
        grid=(N // _CRB,),
        in_specs=[pl.BlockSpec((_CRB, D), lambda i: (i, 0)),
                  pl.BlockSpec((_CRB, D), lambda i: (i, 0))],
        out_specs=pl.BlockSpec((_CRB, D // 2), lambda i: (i, 0)),
        out_shape=jax.ShapeDtypeStruct((N, D // 2), jnp.int32),
    )(node_feat, memory)


_NC, _NS = 2, 16
_NW = _NC * _NS
_CH = 128


@functools.lru_cache(maxsize=None)
def _make_sc_gather(tbc):
    pwn = tbc * K // _NW
    pwh = tbc // _NW
    assert pwn % (2 * _CH) == 0 and pwh % (2 * _CH) == 0

    def body(table, idx_n, idx_h, out_n, out_h, idxl, rows, sg0, sg1,
             ss0, ss1):
        wid = lax.axis_index("s") * _NC + lax.axis_index("c")
        pltpu.sync_copy(idx_n.at[pl.ds(wid * pwn, pwn)], idxl.at[pl.ds(0, pwn)])
        pltpu.sync_copy(idx_h.at[pl.ds(wid * pwh, pwh)],
                        idxl.at[pl.ds(pwn, pwh)])
        sg = (sg0, sg1)
        ss = (ss0, ss1)

        def run(ibase, out_hbm, obase, n_chunks):
            def g_copy(i, b):
                return pltpu.make_async_copy(
                    table.at[idxl.at[pl.ds(ibase + i * _CH, _CH)]],
                    rows.at[b], sg[b])

            def s_copy(i, b):
                return pltpu.make_async_copy(
                    rows.at[b], out_hbm.at[pl.ds(obase + i * _CH, _CH)], ss[b])

            for b in (0, 1):
                g_copy(b, b).start()

            def loop_body(g, carry):
                for b in (0, 1):
                    i = 2 * g + b
                    g_copy(i, b).wait()
                    s_copy(i, b).start()
                for b in (0, 1):
                    i = 2 * g + b

                    def _prefetch(i=i, b=b):
                        s_copy(i, b).wait()
                        g_copy(i + 2, b).start()

                    pl.when(i + 2 < n_chunks)(_prefetch)
                return carry

            lax.fori_loop(0, n_chunks // 2, loop_body, 0)
            for b in (0, 1):
                s_copy(n_chunks - 2 + b, b).wait()

        run(0, out_n, wid * pwn, pwn // _CH)
        run(pwn, out_h, wid * pwh, pwh // _CH)

    return pl.kernel(
        body,
        out_type=(jax.ShapeDtypeStruct((tbc * K, D // 2), jnp.int32),
                  jax.ShapeDtypeStruct((tbc, D // 2), jnp.int32)),
        mesh=plsc.VectorSubcoreMesh(core_axis_name="c", subcore_axis_name="s"),
        compiler_params=pltpu.CompilerParams(use_tc_tiling_on_sc=False),
        scratch_types=[
            pltpu.VMEM((pwn + pwh,), jnp.int32),
            pltpu.VMEM((2, _CH, D // 2), jnp.int32),
            pltpu.SemaphoreType.DMA,
            pltpu.SemaphoreType.DMA,
            pltpu.SemaphoreType.DMA,
            pltpu.SemaphoreType.DMA,
        ],
    )


_R = 128

_INV2PI = 0.15915494309189535
_COSC = (9.9995902495e-01, -1.9730942534e+01, 6.4671443424e+01,
         -8.2390811065e+01, 4.5621052378e+01)


def _fast_cos_pre(r):
    f = r - jnp.floor(r + 0.5)
    u = f * f
    p = jnp.float32(_COSC[4])
    for c in (_COSC[3], _COSC[2], _COSC[1], _COSC[0]):
        p = p * u + jnp.float32(c)
    return p


def _attn_body(h_ref, ngh_ref, nt_ref, ts_ref, tw_ref, tb_ref, wq_ref,
               wk_ref, wv_ref, wm1_ref, bm1_ref, wm2_ref, bm2_ref, out_ref):
    def _unpack(pk):
        lo = lax.bitcast_convert_type(jnp.left_shift(pk, 16), jnp.float32)
        hi = lax.bitcast_convert_type(
            jnp.bitwise_and(pk, jnp.int32(-65536)), jnp.float32)
        return lo, hi

    hlo, hhi = _unpack(h_ref[...])
    nlo3, nhi3 = _unpack(ngh_ref[...])
    nlo = nlo3.reshape(K * _R, D // 2)
    nhi = nhi3.reshape(K * _R, D // 2)
    nt = nt_ref[...]
    ts = ts_ref[...]
    w = tw_ref[...]
    b = tb_ref[...]

    dt = ts - nt
    te = _fast_cos_pre(dt[:, :, None] * w[None, :, :] + b[None, :, :])
    te2 = te.reshape(K * _R, D)

    wk = wk_ref[...]
    wv = wv_ref[...]
    hf = D // 2
    kmat = nlo @ wk[:hf] + nhi @ wk[hf:D] + te2 @ wk[D:]
    vmat = nlo @ wv[:hf] + nhi @ wv[hf:D] + te2 @ wv[D:]

    wq = wq_ref[...]
    scale = 1.0 / (DH ** 0.5)
    q = (hlo @ wq[:hf] + hhi @ wq[hf:D]
         + _fast_cos_pre(b) @ wq[D:]) * scale

    k3 = kmat.reshape(K, _R, D)
    sfull2 = (k3 * q[None, :, :]).reshape(K * _R, D)
    ii = lax.broadcasted_iota(jnp.int32, (D, D), 0)
    jj = lax.broadcasted_iota(jnp.int32, (D, D), 1)
    hsel = ((ii < DH) == (jj < DH)).astype(jnp.float32)
    ebc = jnp.exp(sfull2 @ hsel)
    wei = vmat * ebc
    den = jnp.sum(ebc.reshape(K, _R, D), axis=0)
    o_num = jnp.sum(wei.reshape(K, _R, D), axis=0)
    out_cat = o_num * jnp.reciprocal(den)

    wm1 = wm1_ref[...]
    pre = (out_cat @ wm1[:D] + hlo @ wm1[D:D + hf] + hhi @ wm1[D + hf:]
           + bm1_ref[...])
    out_ref[...] = jnp.maximum(pre, 0.0) @ wm2_ref[...] + bm2_ref[...]


def _attn_call(h, ngh3, nt_t, ts2, tw2, tb2, Wq, Wk, Wv, Wm1, bm1_2, Wm2, bm2_2):
    tbc = h.shape[0]
    full = lambda shape: pl.BlockSpec(shape, lambda i: tuple(0 for _ in shape))
    return pl.pallas_call(
        _attn_body,
        grid=(tbc // _R,),
        in_specs=[
            pl.BlockSpec((_R, D // 2), lambda i: (i, 0)),
            pl.BlockSpec((K, _R, D // 2), lambda i: (0, i, 0)),
            pl.BlockSpec((K, _R), lambda i: (0, i)),
            pl.BlockSpec((1, _R), lambda i: (0, i)),
            full((1, D)), full((1, D)),
            full((2 * D, D)), full((2 * D, D)), full((2 * D, D)),
            full((2 * D, D)), full((1, D)),
            full((D, D)), full((1, D)),
        ],
        out_specs=pl.BlockSpec((_R, D), lambda i: (i, 0)),
        out_shape=jax.ShapeDtypeStruct((tbc, D), jnp.float32),
    )(h, ngh3, nt_t, ts2, tw2, tb2, Wq, Wk, Wv, Wm1, bm1_2, Wm2, bm2_2)


_NCHUNK = 6


def kernel(node_feat, memory, time_w, time_b, Wq, Wk, Wv, Wm1, bm1, Wm2, bm2,
           source_nodes, destination_nodes, negative_nodes, edge_times,
           ngh_idx, ngh_times):
    nodes = jnp.concatenate(
        [source_nodes, destination_nodes, negative_nodes]).astype(jnp.int32)
    ts3 = jnp.concatenate([edge_times, edge_times, edge_times])

    combined = _combine(node_feat, memory)

    idx_t = ngh_idx.astype(jnp.int32).T
    nt_t = ngh_times.T
    tw2 = (time_w * _INV2PI).reshape(1, D)
    tb2 = (time_b * _INV2PI).reshape(1, D)
    bm1_2, bm2_2 = bm1.reshape(1, D), bm2.reshape(1, D)

    tbc = TB // _NCHUNK
    gather = _make_sc_gather(tbc)
    embs = []
    for j in range(_NCHUNK):
        sl = slice(j * tbc, (j + 1) * tbc)
        ngh_flat, h = gather(combined, idx_t[:, sl].reshape(-1), nodes[sl])
        embs.append(_attn_call(
            h, ngh_flat.reshape(K, tbc, D // 2), nt_t[:, sl],
            ts3[sl].reshape(1, tbc), tw2, tb2,
            Wq, Wk, Wv, Wm1, bm1_2, Wm2, bm2_2))
    return jnp.concatenate(embs, axis=0)


# speedup vs baseline: 1.5695x; 1.5695x over previous
import functools

import jax
import jax.numpy as jnp
from jax import lax
from jax.experimental import pallas as pl
from jax.experimental.pallas import tpu as pltpu
from jax.experimental.pallas import tpu_sc as plsc

N = 100000
D = 128
B = 16384
TB = 3 * B
K = 20
H = 2
DH = D // H


_CRB = 4000


def _combine_body(nf_ref, mem_ref, out_ref):
    out_ref[...] = nf_ref[...] + mem_ref[...]


def _combine(node_feat, memory):
    return pl.pallas_call(
        _combine_body,
        grid=(N // _CRB,),
        in_specs=[pl.BlockSpec((_CRB, D), lambda i: (i, 0)),
                  pl.BlockSpec((_CRB, D), lambda i: (i, 0))],
        out_specs=pl.BlockSpec((_CRB, D), lambda i: (i, 0)),
        out_shape=jax.ShapeDtypeStruct((N, D), jnp.float32),
    )(node_feat, memory)


_NC, _NS = 2, 16
_NW = _NC * _NS
_CH = 128


@functools.lru_cache(maxsize=None)
def _make_sc_gather(tbc):
    pwn = tbc * K // _NW
    pwh = tbc // _NW
    assert pwn % (2 * _CH) == 0 and pwh % _CH == 0

    def body(table, idx_n, idx_h, out_n, out_h, idxl, rows, sg0, sg1,
             ss0, ss1):
        wid = lax.axis_index("s") * _NC + lax.axis_index("c")
        pltpu.sync_copy(idx_n.at[pl.ds(wid * pwn, pwn)], idxl.at[pl.ds(0, pwn)])
        pltpu.sync_copy(idx_h.at[pl.ds(wid * pwh, pwh)],
                        idxl.at[pl.ds(pwn, pwh)])
        sg = (sg0, sg1)
        ss = (ss0, ss1)

        def run(ibase, out_hbm, obase, n_chunks):
            def g_copy(i, b):
                return pltpu.make_async_copy(
                    table.at[idxl.at[pl.ds(ibase + i * _CH, _CH)]],
                    rows.at[b], sg[b])

            def s_copy(i, b):
                return pltpu.make_async_copy(
                    rows.at[b], out_hbm.at[pl.ds(obase + i * _CH, _CH)], ss[b])

            if n_chunks == 1:
                g_copy(0, 0).start()
                g_copy(0, 0).wait()
                s_copy(0, 0).start()
                s_copy(0, 0).wait()
                return
            assert n_chunks % 2 == 0

            for b in (0, 1):
                g_copy(b, b).start()

            def loop_body(g, carry):
                for b in (0, 1):
                    i = 2 * g + b
                    g_copy(i, b).wait()
                    s_copy(i, b).start()
                for b in (0, 1):
                    i = 2 * g + b

                    def _prefetch(i=i, b=b):
                        s_copy(i, b).wait()
                        g_copy(i + 2, b).start()

                    pl.when(i + 2 < n_chunks)(_prefetch)
                return carry

            lax.fori_loop(0, n_chunks // 2, loop_body, 0)
            for b in (0, 1):
                s_copy(n_chunks - 2 + b, b).wait()

        run(0, out_n, wid * pwn, pwn // _CH)
        run(pwn, out_h, wid * pwh, pwh // _CH)

    return pl.kernel(
        body,
        out_type=(jax.ShapeDtypeStruct((tbc * K, D), jnp.float32),
                  jax.ShapeDtypeStruct((tbc, D), jnp.float32)),
        mesh=plsc.VectorSubcoreMesh(core_axis_name="c", subcore_axis_name="s"),
        scratch_types=[
            pltpu.VMEM((pwn + pwh,), jnp.int32),
            pltpu.VMEM((2, _CH, D), jnp.float32),
            pltpu.SemaphoreType.DMA,
            pltpu.SemaphoreType.DMA,
            pltpu.SemaphoreType.DMA,
            pltpu.SemaphoreType.DMA,
        ],
    )


_R = 128

_INV2PI = 0.15915494309189535
_COSC = (9.9995902495e-01, -1.9730942534e+01, 6.4671443424e+01,
         -8.2390811065e+01, 4.5621052378e+01)


def _fast_cos_pre(r):
    f = r - jnp.floor(r + 0.5)
    u = f * f
    p = jnp.float32(_COSC[4])
    for c in (_COSC[3], _COSC[2], _COSC[1], _COSC[0]):
        p = p * u + jnp.float32(c)
    return p


def _attn_body(h_ref, ngh_ref, nt_ref, ts_ref, tw_ref, tb_ref, wq_ref,
               wk_ref, wv_ref, wm1_ref, bm1_ref, wm2_ref, bm2_ref, out_ref):
    h = h_ref[...]
    ngh2 = ngh_ref[...].reshape(K * _R, D)
    nt = nt_ref[...]
    ts = ts_ref[...]
    w = tw_ref[...]
    b = tb_ref[...]

    dt = ts - nt
    te = _fast_cos_pre(dt[:, :, None] * w[None, :, :] + b[None, :, :])
    te2 = te.reshape(K * _R, D)

    wk = wk_ref[...]
    wv = wv_ref[...]
    kmat = ngh2 @ wk[:D] + te2 @ wk[D:]
    vmat = ngh2 @ wv[:D] + te2 @ wv[D:]

    wq = wq_ref[...]
    scale = 1.0 / (DH ** 0.5)
    q = (h @ wq[:D] + _fast_cos_pre(b) @ wq[D:]) * scale

    k3 = kmat.reshape(K, _R, D)
    sfull2 = (k3 * q[None, :, :]).reshape(K * _R, D)
    ii = lax.broadcasted_iota(jnp.int32, (D, D), 0)
    jj = lax.broadcasted_iota(jnp.int32, (D, D), 1)
    hsel = ((ii < DH) == (jj < DH)).astype(jnp.float32)
    ebc = jnp.exp(sfull2 @ hsel)
    wei = vmat * ebc
    den = jnp.sum(ebc.reshape(K, _R, D), axis=0)
    o_num = jnp.sum(wei.reshape(K, _R, D), axis=0)
    out_cat = o_num * jnp.reciprocal(den)

    wm1 = wm1_ref[...]
    pre = out_cat @ wm1[:D] + h @ wm1[D:] + bm1_ref[...]
    out_ref[...] = jnp.maximum(pre, 0.0) @ wm2_ref[...] + bm2_ref[...]


def _attn_call(h, ngh3, nt_t, ts2, tw2, tb2, Wq, Wk, Wv, Wm1, bm1_2, Wm2, bm2_2):
    tbc = h.shape[0]
    full = lambda shape: pl.BlockSpec(shape, lambda i: tuple(0 for _ in shape))
    return pl.pallas_call(
        _attn_body,
        grid=(tbc // _R,),
        in_specs=[
            pl.BlockSpec((_R, D), lambda i: (i, 0)),
            pl.BlockSpec((K, _R, D), lambda i: (0, i, 0)),
            pl.BlockSpec((K, _R), lambda i: (0, i)),
            pl.BlockSpec((1, _R), lambda i: (0, i)),
            full((1, D)), full((1, D)),
            full((2 * D, D)), full((2 * D, D)), full((2 * D, D)),
            full((2 * D, D)), full((1, D)),
            full((D, D)), full((1, D)),
        ],
        out_specs=pl.BlockSpec((_R, D), lambda i: (i, 0)),
        out_shape=jax.ShapeDtypeStruct((tbc, D), jnp.float32),
    )(h, ngh3, nt_t, ts2, tw2, tb2, Wq, Wk, Wv, Wm1, bm1_2, Wm2, bm2_2)


_NCHUNK = 12


def kernel(node_feat, memory, time_w, time_b, Wq, Wk, Wv, Wm1, bm1, Wm2, bm2,
           source_nodes, destination_nodes, negative_nodes, edge_times,
           ngh_idx, ngh_times):
    nodes = jnp.concatenate(
        [source_nodes, destination_nodes, negative_nodes]).astype(jnp.int32)
    ts3 = jnp.concatenate([edge_times, edge_times, edge_times])

    combined = _combine(node_feat, memory)

    idx_t = ngh_idx.astype(jnp.int32).T
    nt_t = ngh_times.T
    tw2 = (time_w * _INV2PI).reshape(1, D)
    tb2 = (time_b * _INV2PI).reshape(1, D)
    bm1_2, bm2_2 = bm1.reshape(1, D), bm2.reshape(1, D)

    tbc = TB // _NCHUNK
    gather = _make_sc_gather(tbc)
    embs = []
    for j in range(_NCHUNK):
        sl = slice(j * tbc, (j + 1) * tbc)
        ngh_flat, h = gather(combined, idx_t[:, sl].reshape(-1), nodes[sl])
        embs.append(_attn_call(
            h, ngh_flat.reshape(K, tbc, D), nt_t[:, sl],
            ts3[sl].reshape(1, tbc), tw2, tb2,
            Wq, Wk, Wv, Wm1, bm1_2, Wm2, bm2_2))
    return jnp.concatenate(embs, axis=0)


# speedup vs baseline: 1.8219x; 1.1608x over previous
import functools

import jax
import jax.numpy as jnp
from jax import lax
from jax.experimental import pallas as pl
from jax.experimental.pallas import tpu as pltpu
from jax.experimental.pallas import tpu_sc as plsc

N = 100000
D = 128
B = 16384
TB = 3 * B
K = 20
H = 2
DH = D // H


_CRB = 4000


def _combine_body(nf_ref, mem_ref, out_ref):
    out_ref[...] = nf_ref[...] + mem_ref[...]


def _combine(node_feat, memory):
    return pl.pallas_call(
        _combine_body,
        grid=(N // _CRB,),
        in_specs=[pl.BlockSpec((_CRB, D), lambda i: (i, 0)),
                  pl.BlockSpec((_CRB, D), lambda i: (i, 0))],
        out_specs=pl.BlockSpec((_CRB, D), lambda i: (i, 0)),
        out_shape=jax.ShapeDtypeStruct((N, D), jnp.float32),
    )(node_feat, memory)


_NC, _NS = 2, 16
_NW = _NC * _NS
_CH = 128


@functools.lru_cache(maxsize=None)
def _make_sc_gather(tbc):
    pwn = tbc * K // _NW
    pwh = tbc // _NW
    assert pwn % (2 * _CH) == 0 and pwh % _CH == 0

    def body(table, idx_n, idx_h, out_n, out_h, idxl, rows, sg0, sg1,
             ss0, ss1):
        wid = lax.axis_index("s") * _NC + lax.axis_index("c")
        pltpu.sync_copy(idx_n.at[pl.ds(wid * pwn, pwn)], idxl.at[pl.ds(0, pwn)])
        pltpu.sync_copy(idx_h.at[pl.ds(wid * pwh, pwh)],
                        idxl.at[pl.ds(pwn, pwh)])
        sg = (sg0, sg1)
        ss = (ss0, ss1)

        def run(ibase, out_hbm, obase, n_chunks):
            def g_copy(i, b):
                return pltpu.make_async_copy(
                    table.at[idxl.at[pl.ds(ibase + i * _CH, _CH)]],
                    rows.at[b], sg[b])

            def s_copy(i, b):
                return pltpu.make_async_copy(
                    rows.at[b], out_hbm.at[pl.ds(obase + i * _CH, _CH)], ss[b])

            if n_chunks == 1:
                g_copy(0, 0).start()
                g_copy(0, 0).wait()
                s_copy(0, 0).start()
                s_copy(0, 0).wait()
                return
            assert n_chunks % 2 == 0

            for b in (0, 1):
                g_copy(b, b).start()

            def loop_body(g, carry):
                for b in (0, 1):
                    i = 2 * g + b
                    g_copy(i, b).wait()
                    s_copy(i, b).start()
                for b in (0, 1):
                    i = 2 * g + b

                    def _prefetch(i=i, b=b):
                        s_copy(i, b).wait()
                        g_copy(i + 2, b).start()

                    pl.when(i + 2 < n_chunks)(_prefetch)
                return carry

            lax.fori_loop(0, n_chunks // 2, loop_body, 0)
            for b in (0, 1):
                s_copy(n_chunks - 2 + b, b).wait()

        run(0, out_n, wid * pwn, pwn // _CH)
        run(pwn, out_h, wid * pwh, pwh // _CH)

    return pl.kernel(
        body,
        out_type=(jax.ShapeDtypeStruct((tbc * K, D), jnp.float32),
                  jax.ShapeDtypeStruct((tbc, D), jnp.float32)),
        mesh=plsc.VectorSubcoreMesh(core_axis_name="c", subcore_axis_name="s"),
        scratch_types=[
            pltpu.VMEM((pwn + pwh,), jnp.int32),
            pltpu.VMEM((2, _CH, D), jnp.float32),
            pltpu.SemaphoreType.DMA,
            pltpu.SemaphoreType.DMA,
            pltpu.SemaphoreType.DMA,
            pltpu.SemaphoreType.DMA,
        ],
    )


_R = 256

_INV2PI = 0.15915494309189535
_COSC = (9.9995902495e-01, -1.9730942534e+01, 6.4671443424e+01,
         -8.2390811065e+01, 4.5621052378e+01)


def _fast_cos_pre(r):
    f = r - jnp.floor(r + 0.5)
    u = f * f
    p = jnp.float32(_COSC[4])
    for c in (_COSC[3], _COSC[2], _COSC[1], _COSC[0]):
        p = p * u + jnp.float32(c)
    return p


def _attn_body(h_ref, ngh_ref, nt_ref, ts_ref, tw_ref, tb_ref, wq_ref,
               wk_ref, wv_ref, wm1_ref, bm1_ref, wm2_ref, bm2_ref, out_ref):
    h = h_ref[...]
    ngh2 = ngh_ref[...].reshape(K * _R, D)
    nt = nt_ref[...]
    ts = ts_ref[...]
    w = tw_ref[...]
    b = tb_ref[...]

    dt = ts - nt
    te = _fast_cos_pre(dt[:, :, None] * w[None, :, :] + b[None, :, :])
    te2 = te.reshape(K * _R, D)

    wk = wk_ref[...]
    wv = wv_ref[...]
    kmat = ngh2 @ wk[:D] + te2 @ wk[D:]
    vmat = ngh2 @ wv[:D] + te2 @ wv[D:]

    wq = wq_ref[...]
    scale = 1.0 / (DH ** 0.5)
    q = (h @ wq[:D] + _fast_cos_pre(b) @ wq[D:]) * scale

    k3 = kmat.reshape(K, _R, D)
    sfull2 = (k3 * q[None, :, :]).reshape(K * _R, D)
    ii = lax.broadcasted_iota(jnp.int32, (D, D), 0)
    jj = lax.broadcasted_iota(jnp.int32, (D, D), 1)
    hsel = ((ii < DH) == (jj < DH)).astype(jnp.float32)
    ebc = jnp.exp(sfull2 @ hsel)
    wei = vmat * ebc
    den = jnp.sum(ebc.reshape(K, _R, D), axis=0)
    o_num = jnp.sum(wei.reshape(K, _R, D), axis=0)
    out_cat = o_num * jnp.reciprocal(den)

    wm1 = wm1_ref[...]
    pre = out_cat @ wm1[:D] + h @ wm1[D:] + bm1_ref[...]
    out_ref[...] = jnp.maximum(pre, 0.0) @ wm2_ref[...] + bm2_ref[...]


def _attn_call(h, ngh3, nt_t, ts2, tw2, tb2, Wq, Wk, Wv, Wm1, bm1_2, Wm2, bm2_2):
    tbc = h.shape[0]
    full = lambda shape: pl.BlockSpec(shape, lambda i: tuple(0 for _ in shape))
    return pl.pallas_call(
        _attn_body,
        grid=(tbc // _R,),
        in_specs=[
            pl.BlockSpec((_R, D), lambda i: (i, 0)),
            pl.BlockSpec((K, _R, D), lambda i: (0, i, 0)),
            pl.BlockSpec((K, _R), lambda i: (0, i)),
            pl.BlockSpec((1, _R), lambda i: (0, i)),
            full((1, D)), full((1, D)),
            full((2 * D, D)), full((2 * D, D)), full((2 * D, D)),
            full((2 * D, D)), full((1, D)),
            full((D, D)), full((1, D)),
        ],
        out_specs=pl.BlockSpec((_R, D), lambda i: (i, 0)),
        out_shape=jax.ShapeDtypeStruct((tbc, D), jnp.float32),
    )(h, ngh3, nt_t, ts2, tw2, tb2, Wq, Wk, Wv, Wm1, bm1_2, Wm2, bm2_2)


_NCHUNK = 12


def kernel(node_feat, memory, time_w, time_b, Wq, Wk, Wv, Wm1, bm1, Wm2, bm2,
           source_nodes, destination_nodes, negative_nodes, edge_times,
           ngh_idx, ngh_times):
    nodes = jnp.concatenate(
        [source_nodes, destination_nodes, negative_nodes]).astype(jnp.int32)
    ts3 = jnp.concatenate([edge_times, edge_times, edge_times])

    combined = _combine(node_feat, memory)

    idx_t = ngh_idx.astype(jnp.int32).T
    nt_t = ngh_times.T
    tw2 = (time_w * _INV2PI).reshape(1, D)
    tb2 = (time_b * _INV2PI).reshape(1, D)
    bm1_2, bm2_2 = bm1.reshape(1, D), bm2.reshape(1, D)

    tbc = TB // _NCHUNK
    gather = _make_sc_gather(tbc)
    embs = []
    for j in range(_NCHUNK):
        sl = slice(j * tbc, (j + 1) * tbc)
        ngh_flat, h = gather(combined, idx_t[:, sl].reshape(-1), nodes[sl])
        embs.append(_attn_call(
            h, ngh_flat.reshape(K, tbc, D), nt_t[:, sl],
            ts3[sl].reshape(1, tbc), tw2, tb2,
            Wq, Wk, Wv, Wm1, bm1_2, Wm2, bm2_2))
    return jnp.concatenate(embs, axis=0)


# speedup vs baseline: 1.9300x; 1.0593x over previous
import functools

import jax
import jax.numpy as jnp
from jax import lax
from jax.experimental import pallas as pl
from jax.experimental.pallas import tpu as pltpu
from jax.experimental.pallas import tpu_sc as plsc

N = 100000
D = 128
B = 16384
TB = 3 * B
K = 20
H = 2
DH = D // H


_CRB = 4000


def _combine_body(nf_ref, mem_ref, out_ref):
    out_ref[...] = nf_ref[...] + mem_ref[...]


def _combine(node_feat, memory):
    return pl.pallas_call(
        _combine_body,
        grid=(N // _CRB,),
        in_specs=[pl.BlockSpec((_CRB, D), lambda i: (i, 0)),
                  pl.BlockSpec((_CRB, D), lambda i: (i, 0))],
        out_specs=pl.BlockSpec((_CRB, D), lambda i: (i, 0)),
        out_shape=jax.ShapeDtypeStruct((N, D), jnp.float32),
    )(node_feat, memory)


_NC, _NS = 2, 16
_NW = _NC * _NS
_CH = 128


@functools.lru_cache(maxsize=None)
def _make_sc_gather(tbc):
    pwn = tbc * K // _NW
    pwh = tbc // _NW
    assert pwn % (2 * _CH) == 0 and pwh % _CH == 0

    def body(table, idx_n, idx_h, out_n, out_h, idxl, rows, sg0, sg1,
             ss0, ss1):
        wid = lax.axis_index("s") * _NC + lax.axis_index("c")
        pltpu.sync_copy(idx_n.at[pl.ds(wid * pwn, pwn)], idxl.at[pl.ds(0, pwn)])
        pltpu.sync_copy(idx_h.at[pl.ds(wid * pwh, pwh)],
                        idxl.at[pl.ds(pwn, pwh)])
        sg = (sg0, sg1)
        ss = (ss0, ss1)

        def run(ibase, out_hbm, obase, n_chunks):
            def g_copy(i, b):
                return pltpu.make_async_copy(
                    table.at[idxl.at[pl.ds(ibase + i * _CH, _CH)]],
                    rows.at[b], sg[b])

            def s_copy(i, b):
                return pltpu.make_async_copy(
                    rows.at[b], out_hbm.at[pl.ds(obase + i * _CH, _CH)], ss[b])

            if n_chunks == 1:
                g_copy(0, 0).start()
                g_copy(0, 0).wait()
                s_copy(0, 0).start()
                s_copy(0, 0).wait()
                return
            assert n_chunks % 2 == 0

            for b in (0, 1):
                g_copy(b, b).start()

            def loop_body(g, carry):
                for b in (0, 1):
                    i = 2 * g + b
                    g_copy(i, b).wait()
                    s_copy(i, b).start()
                for b in (0, 1):
                    i = 2 * g + b

                    def _prefetch(i=i, b=b):
                        s_copy(i, b).wait()
                        g_copy(i + 2, b).start()

                    pl.when(i + 2 < n_chunks)(_prefetch)
                return carry

            lax.fori_loop(0, n_chunks // 2, loop_body, 0)
            for b in (0, 1):
                s_copy(n_chunks - 2 + b, b).wait()

        run(0, out_n, wid * pwn, pwn // _CH)
        run(pwn, out_h, wid * pwh, pwh // _CH)

    return pl.kernel(
        body,
        out_type=(jax.ShapeDtypeStruct((tbc * K, D), jnp.float32),
                  jax.ShapeDtypeStruct((tbc, D), jnp.float32)),
        mesh=plsc.VectorSubcoreMesh(core_axis_name="c", subcore_axis_name="s"),
        scratch_types=[
            pltpu.VMEM((pwn + pwh,), jnp.int32),
            pltpu.VMEM((2, _CH, D), jnp.float32),
            pltpu.SemaphoreType.DMA,
            pltpu.SemaphoreType.DMA,
            pltpu.SemaphoreType.DMA,
            pltpu.SemaphoreType.DMA,
        ],
    )


_R = 512

_INV2PI = 0.15915494309189535
_COSC = (9.9995902495e-01, -1.9730942534e+01, 6.4671443424e+01,
         -8.2390811065e+01, 4.5621052378e+01)


def _fast_cos_pre(r):
    f = r - jnp.floor(r + 0.5)
    u = f * f
    p = jnp.float32(_COSC[4])
    for c in (_COSC[3], _COSC[2], _COSC[1], _COSC[0]):
        p = p * u + jnp.float32(c)
    return p


def _attn_body(h_ref, ngh_ref, nt_ref, ts_ref, tw_ref, tb_ref, wq_ref,
               wk_ref, wv_ref, wm1_ref, bm1_ref, wm2_ref, bm2_ref, out_ref):
    h = h_ref[...]
    ngh2 = ngh_ref[...].reshape(K * _R, D)
    nt = nt_ref[...]
    ts = ts_ref[...]
    w = tw_ref[...]
    b = tb_ref[...]

    dt = ts - nt
    te = _fast_cos_pre(dt[:, :, None] * w[None, :, :] + b[None, :, :])
    te2 = te.reshape(K * _R, D)

    wk = wk_ref[...]
    wv = wv_ref[...]
    kmat = ngh2 @ wk[:D] + te2 @ wk[D:]
    vmat = ngh2 @ wv[:D] + te2 @ wv[D:]

    wq = wq_ref[...]
    scale = 1.0 / (DH ** 0.5)
    q = (h @ wq[:D] + _fast_cos_pre(b) @ wq[D:]) * scale

    k3 = kmat.reshape(K, _R, D)
    sfull2 = (k3 * q[None, :, :]).reshape(K * _R, D)
    ii = lax.broadcasted_iota(jnp.int32, (D, D), 0)
    jj = lax.broadcasted_iota(jnp.int32, (D, D), 1)
    hsel = ((ii < DH) == (jj < DH)).astype(jnp.float32)
    ebc = jnp.exp(sfull2 @ hsel)
    wei = vmat * ebc
    den = jnp.sum(ebc.reshape(K, _R, D), axis=0)
    o_num = jnp.sum(wei.reshape(K, _R, D), axis=0)
    out_cat = o_num * jnp.reciprocal(den)

    wm1 = wm1_ref[...]
    pre = out_cat @ wm1[:D] + h @ wm1[D:] + bm1_ref[...]
    out_ref[...] = jnp.maximum(pre, 0.0) @ wm2_ref[...] + bm2_ref[...]


def _attn_call(h, ngh3, nt_t, ts2, tw2, tb2, Wq, Wk, Wv, Wm1, bm1_2, Wm2, bm2_2):
    tbc = h.shape[0]
    full = lambda shape: pl.BlockSpec(shape, lambda i: tuple(0 for _ in shape))
    return pl.pallas_call(
        _attn_body,
        grid=(tbc // _R,),
        in_specs=[
            pl.BlockSpec((_R, D), lambda i: (i, 0)),
            pl.BlockSpec((K, _R, D), lambda i: (0, i, 0)),
            pl.BlockSpec((K, _R), lambda i: (0, i)),
            pl.BlockSpec((1, _R), lambda i: (0, i)),
            full((1, D)), full((1, D)),
            full((2 * D, D)), full((2 * D, D)), full((2 * D, D)),
            full((2 * D, D)), full((1, D)),
            full((D, D)), full((1, D)),
        ],
        out_specs=pl.BlockSpec((_R, D), lambda i: (i, 0)),
        out_shape=jax.ShapeDtypeStruct((tbc, D), jnp.float32),
    )(h, ngh3, nt_t, ts2, tw2, tb2, Wq, Wk, Wv, Wm1, bm1_2, Wm2, bm2_2)


_NCHUNK = 12


def kernel(node_feat, memory, time_w, time_b, Wq, Wk, Wv, Wm1, bm1, Wm2, bm2,
           source_nodes, destination_nodes, negative_nodes, edge_times,
           ngh_idx, ngh_times):
    nodes = jnp.concatenate(
        [source_nodes, destination_nodes, negative_nodes]).astype(jnp.int32)
    ts3 = jnp.concatenate([edge_times, edge_times, edge_times])

    combined = _combine(node_feat, memory)

    idx_t = ngh_idx.astype(jnp.int32).T
    nt_t = ngh_times.T
    tw2 = (time_w * _INV2PI).reshape(1, D)
    tb2 = (time_b * _INV2PI).reshape(1, D)
    bm1_2, bm2_2 = bm1.reshape(1, D), bm2.reshape(1, D)

    tbc = TB // _NCHUNK
    gather = _make_sc_gather(tbc)
    embs = []
    for j in range(_NCHUNK):
        sl = slice(j * tbc, (j + 1) * tbc)
        ngh_flat, h = gather(combined, idx_t[:, sl].reshape(-1), nodes[sl])
        embs.append(_attn_call(
            h, ngh_flat.reshape(K, tbc, D), nt_t[:, sl],
            ts3[sl].reshape(1, tbc), tw2, tb2,
            Wq, Wk, Wv, Wm1, bm1_2, Wm2, bm2_2))
    return jnp.concatenate(embs, axis=0)


# speedup vs baseline: 1.9387x; 1.0045x over previous
import functools

import jax
import jax.numpy as jnp
from jax import lax
from jax.experimental import pallas as pl
from jax.experimental.pallas import tpu as pltpu
from jax.experimental.pallas import tpu_sc as plsc

N = 100000
D = 128
B = 16384
TB = 3 * B
K = 20
H = 2
DH = D // H


_CRB = 4000


def _combine_body(nf_ref, mem_ref, out_ref):
    out_ref[...] = nf_ref[...] + mem_ref[...]


def _combine(node_feat, memory):
    return pl.pallas_call(
        _combine_body,
        grid=(N // _CRB,),
        in_specs=[pl.BlockSpec((_CRB, D), lambda i: (i, 0)),
                  pl.BlockSpec((_CRB, D), lambda i: (i, 0))],
        out_specs=pl.BlockSpec((_CRB, D), lambda i: (i, 0)),
        out_shape=jax.ShapeDtypeStruct((N, D), jnp.float32),
    )(node_feat, memory)


_NC, _NS = 2, 16
_NW = _NC * _NS
_CH = 128


@functools.lru_cache(maxsize=None)
def _make_sc_gather(tbc):
    pwn = tbc * K // _NW
    pwh = tbc // _NW
    assert pwn % (2 * _CH) == 0 and pwh % _CH == 0

    def body(table, idx_n, idx_h, out_n, out_h, idxl, rows, sg0, sg1,
             ss0, ss1):
        wid = lax.axis_index("s") * _NC + lax.axis_index("c")
        pltpu.sync_copy(idx_n.at[pl.ds(wid * pwn, pwn)], idxl.at[pl.ds(0, pwn)])
        pltpu.sync_copy(idx_h.at[pl.ds(wid * pwh, pwh)],
                        idxl.at[pl.ds(pwn, pwh)])
        sg = (sg0, sg1)
        ss = (ss0, ss1)

        def run(ibase, out_hbm, obase, n_chunks):
            def g_copy(i, b):
                return pltpu.make_async_copy(
                    table.at[idxl.at[pl.ds(ibase + i * _CH, _CH)]],
                    rows.at[b], sg[b])

            def s_copy(i, b):
                return pltpu.make_async_copy(
                    rows.at[b], out_hbm.at[pl.ds(obase + i * _CH, _CH)], ss[b])

            if n_chunks == 1:
                g_copy(0, 0).start()
                g_copy(0, 0).wait()
                s_copy(0, 0).start()
                s_copy(0, 0).wait()
                return
            assert n_chunks % 2 == 0

            for b in (0, 1):
                g_copy(b, b).start()

            def loop_body(g, carry):
                for b in (0, 1):
                    i = 2 * g + b
                    g_copy(i, b).wait()
                    s_copy(i, b).start()
                for b in (0, 1):
                    i = 2 * g + b

                    def _prefetch(i=i, b=b):
                        s_copy(i, b).wait()
                        g_copy(i + 2, b).start()

                    pl.when(i + 2 < n_chunks)(_prefetch)
                return carry

            lax.fori_loop(0, n_chunks // 2, loop_body, 0)
            for b in (0, 1):
                s_copy(n_chunks - 2 + b, b).wait()

        run(0, out_n, wid * pwn, pwn // _CH)
        run(pwn, out_h, wid * pwh, pwh // _CH)

    return pl.kernel(
        body,
        out_type=(jax.ShapeDtypeStruct((tbc * K, D), jnp.float32),
                  jax.ShapeDtypeStruct((tbc, D), jnp.float32)),
        mesh=plsc.VectorSubcoreMesh(core_axis_name="c", subcore_axis_name="s"),
        scratch_types=[
            pltpu.VMEM((pwn + pwh,), jnp.int32),
            pltpu.VMEM((2, _CH, D), jnp.float32),
            pltpu.SemaphoreType.DMA,
            pltpu.SemaphoreType.DMA,
            pltpu.SemaphoreType.DMA,
            pltpu.SemaphoreType.DMA,
        ],
    )


_R = 512

_INV2PI = 0.15915494309189535
_COSC = (9.9995902495e-01, -1.9730942534e+01, 6.4671443424e+01,
         -8.2390811065e+01, 4.5621052378e+01)


def _fast_cos_pre(r):
    f = r - jnp.floor(r + 0.5)
    u = f * f
    p = jnp.float32(_COSC[4])
    for c in (_COSC[3], _COSC[2], _COSC[1], _COSC[0]):
        p = p * u + jnp.float32(c)
    return p


def _attn_body(h_ref, ngh_ref, nt_ref, ts_ref, tw_ref, tb_ref, wq_ref,
               wk_ref, wv_ref, wm1_ref, bm1_ref, wm2_ref, bm2_ref, out_ref):
    h = h_ref[...]
    ngh2 = ngh_ref[...].reshape(K * _R, D)
    nt = nt_ref[...]
    ts = ts_ref[...]
    w = tw_ref[...]
    b = tb_ref[...]

    dt = ts - nt
    te = _fast_cos_pre(dt[:, :, None] * w[None, :, :] + b[None, :, :])
    te2 = te.reshape(K * _R, D)

    wk = wk_ref[...]
    wv = wv_ref[...]
    kmat = ngh2 @ wk[:D] + te2 @ wk[D:]
    vmat = ngh2 @ wv[:D] + te2 @ wv[D:]

    wq = wq_ref[...]
    scale = 1.0 / (DH ** 0.5)
    q = (h @ wq[:D] + _fast_cos_pre(b) @ wq[D:]) * scale

    k3 = kmat.reshape(K, _R, D)
    sfull2 = (k3 * q[None, :, :]).reshape(K * _R, D)
    ii = lax.broadcasted_iota(jnp.int32, (D, D), 0)
    jj = lax.broadcasted_iota(jnp.int32, (D, D), 1)
    hsel = ((ii < DH) == (jj < DH)).astype(jnp.float32)
    ebc = jnp.exp(sfull2 @ hsel)
    wei = vmat * ebc
    den = jnp.sum(ebc.reshape(K, _R, D), axis=0)
    o_num = jnp.sum(wei.reshape(K, _R, D), axis=0)
    out_cat = o_num * jnp.reciprocal(den)

    wm1 = wm1_ref[...]
    pre = out_cat @ wm1[:D] + h @ wm1[D:] + bm1_ref[...]
    out_ref[...] = jnp.maximum(pre, 0.0) @ wm2_ref[...] + bm2_ref[...]


def _attn_call(h, ngh3, nt_t, ts2, tw2, tb2, Wq, Wk, Wv, Wm1, bm1_2, Wm2, bm2_2):
    tbc = h.shape[0]
    full = lambda shape: pl.BlockSpec(shape, lambda i: tuple(0 for _ in shape))
    return pl.pallas_call(
        _attn_body,
        grid=(tbc // _R,),
        in_specs=[
            pl.BlockSpec((_R, D), lambda i: (i, 0)),
            pl.BlockSpec((K, _R, D), lambda i: (0, i, 0)),
            pl.BlockSpec((K, _R), lambda i: (0, i)),
            pl.BlockSpec((1, _R), lambda i: (0, i)),
            full((1, D)), full((1, D)),
            full((2 * D, D)), full((2 * D, D)), full((2 * D, D)),
            full((2 * D, D)), full((1, D)),
            full((D, D)), full((1, D)),
        ],
        out_specs=pl.BlockSpec((_R, D), lambda i: (i, 0)),
        out_shape=jax.ShapeDtypeStruct((tbc, D), jnp.float32),
    )(h, ngh3, nt_t, ts2, tw2, tb2, Wq, Wk, Wv, Wm1, bm1_2, Wm2, bm2_2)


_NCHUNK = 6


def kernel(node_feat, memory, time_w, time_b, Wq, Wk, Wv, Wm1, bm1, Wm2, bm2,
           source_nodes, destination_nodes, negative_nodes, edge_times,
           ngh_idx, ngh_times):
    nodes = jnp.concatenate(
        [source_nodes, destination_nodes, negative_nodes]).astype(jnp.int32)
    ts3 = jnp.concatenate([edge_times, edge_times, edge_times])

    combined = _combine(node_feat, memory)

    idx_t = ngh_idx.astype(jnp.int32).T
    nt_t = ngh_times.T
    tw2 = (time_w * _INV2PI).reshape(1, D)
    tb2 = (time_b * _INV2PI).reshape(1, D)
    bm1_2, bm2_2 = bm1.reshape(1, D), bm2.reshape(1, D)

    tbc = TB // _NCHUNK
    gather = _make_sc_gather(tbc)
    embs = []
    for j in range(_NCHUNK):
        sl = slice(j * tbc, (j + 1) * tbc)
        ngh_flat, h = gather(combined, idx_t[:, sl].reshape(-1), nodes[sl])
        embs.append(_attn_call(
            h, ngh_flat.reshape(K, tbc, D), nt_t[:, sl],
            ts3[sl].reshape(1, tbc), tw2, tb2,
            Wq, Wk, Wv, Wm1, bm1_2, Wm2, bm2_2))
    return jnp.concatenate(embs, axis=0)


# speedup vs baseline: 1.9599x; 1.0109x over previous
import functools

import jax
import jax.numpy as jnp
from jax import lax
from jax.experimental import pallas as pl
from jax.experimental.pallas import tpu as pltpu
from jax.experimental.pallas import tpu_sc as plsc

N = 100000
D = 128
B = 16384
TB = 3 * B
K = 20
H = 2
DH = D // H


_CRB = 4000


def _combine_body(nf_ref, mem_ref, out_ref):
    out_ref[...] = nf_ref[...] + mem_ref[...]


def _combine(node_feat, memory):
    return pl.pallas_call(
        _combine_body,
        grid=(N // _CRB,),
        in_specs=[pl.BlockSpec((_CRB, D), lambda i: (i, 0)),
                  pl.BlockSpec((_CRB, D), lambda i: (i, 0))],
        out_specs=pl.BlockSpec((_CRB, D), lambda i: (i, 0)),
        out_shape=jax.ShapeDtypeStruct((N, D), jnp.float32),
    )(node_feat, memory)


_NC, _NS = 2, 16
_NW = _NC * _NS
_CH = 128


@functools.lru_cache(maxsize=None)
def _make_sc_gather(tbc):
    pwn = tbc * K // _NW
    pwh = tbc // _NW
    assert pwn % (2 * _CH) == 0 and pwh % _CH == 0

    def body(table, idx_n, idx_h, out_n, out_h, idxl, rows, sg0, sg1,
             ss0, ss1):
        wid = lax.axis_index("s") * _NC + lax.axis_index("c")
        pltpu.sync_copy(idx_n.at[pl.ds(wid * pwn, pwn)], idxl.at[pl.ds(0, pwn)])
        pltpu.sync_copy(idx_h.at[pl.ds(wid * pwh, pwh)],
                        idxl.at[pl.ds(pwn, pwh)])
        sg = (sg0, sg1)
        ss = (ss0, ss1)

        def run(ibase, out_hbm, obase, n_chunks):
            def g_copy(i, b):
                return pltpu.make_async_copy(
                    table.at[idxl.at[pl.ds(ibase + i * _CH, _CH)]],
                    rows.at[b], sg[b])

            def s_copy(i, b):
                return pltpu.make_async_copy(
                    rows.at[b], out_hbm.at[pl.ds(obase + i * _CH, _CH)], ss[b])

            if n_chunks == 1:
                g_copy(0, 0).start()
                g_copy(0, 0).wait()
                s_copy(0, 0).start()
                s_copy(0, 0).wait()
                return
            assert n_chunks % 2 == 0

            for b in (0, 1):
                g_copy(b, b).start()

            def loop_body(g, carry):
                for b in (0, 1):
                    i = 2 * g + b
                    g_copy(i, b).wait()
                    s_copy(i, b).start()
                for b in (0, 1):
                    i = 2 * g + b

                    def _prefetch(i=i, b=b):
                        s_copy(i, b).wait()
                        g_copy(i + 2, b).start()

                    pl.when(i + 2 < n_chunks)(_prefetch)
                return carry

            lax.fori_loop(0, n_chunks // 2, loop_body, 0)
            for b in (0, 1):
                s_copy(n_chunks - 2 + b, b).wait()

        run(0, out_n, wid * pwn, pwn // _CH)
        run(pwn, out_h, wid * pwh, pwh // _CH)

    return pl.kernel(
        body,
        out_type=(jax.ShapeDtypeStruct((tbc * K, D), jnp.float32),
                  jax.ShapeDtypeStruct((tbc, D), jnp.float32)),
        mesh=plsc.VectorSubcoreMesh(core_axis_name="c", subcore_axis_name="s"),
        scratch_types=[
            pltpu.VMEM((pwn + pwh,), jnp.int32),
            pltpu.VMEM((2, _CH, D), jnp.float32),
            pltpu.SemaphoreType.DMA,
            pltpu.SemaphoreType.DMA,
            pltpu.SemaphoreType.DMA,
            pltpu.SemaphoreType.DMA,
        ],
    )


_R = 1024

_INV2PI = 0.15915494309189535
_COSC = (9.9995902495e-01, -1.9730942534e+01, 6.4671443424e+01,
         -8.2390811065e+01, 4.5621052378e+01)


def _fast_cos_pre(r):
    f = r - jnp.floor(r + 0.5)
    u = f * f
    p = jnp.float32(_COSC[4])
    for c in (_COSC[3], _COSC[2], _COSC[1], _COSC[0]):
        p = p * u + jnp.float32(c)
    return p


def _attn_body(h_ref, ngh_ref, nt_ref, ts_ref, tw_ref, tb_ref, wq_ref,
               wk_ref, wv_ref, wm1_ref, bm1_ref, wm2_ref, bm2_ref, out_ref):
    h = h_ref[...]
    ngh2 = ngh_ref[...].reshape(K * _R, D)
    nt = nt_ref[...]
    ts = ts_ref[...]
    w = tw_ref[...]
    b = tb_ref[...]

    dt = ts - nt
    te = _fast_cos_pre(dt[:, :, None] * w[None, :, :] + b[None, :, :])
    te2 = te.reshape(K * _R, D)

    wk = wk_ref[...]
    wv = wv_ref[...]
    kmat = ngh2 @ wk[:D] + te2 @ wk[D:]
    vmat = ngh2 @ wv[:D] + te2 @ wv[D:]

    wq = wq_ref[...]
    scale = 1.0 / (DH ** 0.5)
    q = (h @ wq[:D] + _fast_cos_pre(b) @ wq[D:]) * scale

    k3 = kmat.reshape(K, _R, D)
    sfull2 = (k3 * q[None, :, :]).reshape(K * _R, D)
    ii = lax.broadcasted_iota(jnp.int32, (D, D), 0)
    jj = lax.broadcasted_iota(jnp.int32, (D, D), 1)
    hsel = ((ii < DH) == (jj < DH)).astype(jnp.float32)
    ebc = jnp.exp(sfull2 @ hsel)
    wei = vmat * ebc
    den = jnp.sum(ebc.reshape(K, _R, D), axis=0)
    o_num = jnp.sum(wei.reshape(K, _R, D), axis=0)
    out_cat = o_num * jnp.reciprocal(den)

    wm1 = wm1_ref[...]
    pre = out_cat @ wm1[:D] + h @ wm1[D:] + bm1_ref[...]
    out_ref[...] = jnp.maximum(pre, 0.0) @ wm2_ref[...] + bm2_ref[...]


def _attn_call(h, ngh3, nt_t, ts2, tw2, tb2, Wq, Wk, Wv, Wm1, bm1_2, Wm2, bm2_2):
    tbc = h.shape[0]
    full = lambda shape: pl.BlockSpec(shape, lambda i: tuple(0 for _ in shape))
    return pl.pallas_call(
        _attn_body,
        grid=(tbc // _R,),
        in_specs=[
            pl.BlockSpec((_R, D), lambda i: (i, 0)),
            pl.BlockSpec((K, _R, D), lambda i: (0, i, 0)),
            pl.BlockSpec((K, _R), lambda i: (0, i)),
            pl.BlockSpec((1, _R), lambda i: (0, i)),
            full((1, D)), full((1, D)),
            full((2 * D, D)), full((2 * D, D)), full((2 * D, D)),
            full((2 * D, D)), full((1, D)),
            full((D, D)), full((1, D)),
        ],
        out_specs=pl.BlockSpec((_R, D), lambda i: (i, 0)),
        out_shape=jax.ShapeDtypeStruct((tbc, D), jnp.float32),
    )(h, ngh3, nt_t, ts2, tw2, tb2, Wq, Wk, Wv, Wm1, bm1_2, Wm2, bm2_2)


_NCHUNK = 6


def kernel(node_feat, memory, time_w, time_b, Wq, Wk, Wv, Wm1, bm1, Wm2, bm2,
           source_nodes, destination_nodes, negative_nodes, edge_times,
           ngh_idx, ngh_times):
    nodes = jnp.concatenate(
        [source_nodes, destination_nodes, negative_nodes]).astype(jnp.int32)
    ts3 = jnp.concatenate([edge_times, edge_times, edge_times])

    combined = _combine(node_feat, memory)

    idx_t = ngh_idx.astype(jnp.int32).T
    nt_t = ngh_times.T
    tw2 = (time_w * _INV2PI).reshape(1, D)
    tb2 = (time_b * _INV2PI).reshape(1, D)
    bm1_2, bm2_2 = bm1.reshape(1, D), bm2.reshape(1, D)

    tbc = TB // _NCHUNK
    gather = _make_sc_gather(tbc)
    embs = []
    for j in range(_NCHUNK):
        sl = slice(j * tbc, (j + 1) * tbc)
        ngh_flat, h = gather(combined, idx_t[:, sl].reshape(-1), nodes[sl])
        embs.append(_attn_call(
            h, ngh_flat.reshape(K, tbc, D), nt_t[:, sl],
            ts3[sl].reshape(1, tbc), tw2, tb2,
            Wq, Wk, Wv, Wm1, bm1_2, Wm2, bm2_2))
    return jnp.concatenate(embs, axis=0)
